# baseline (speedup 1.0000x reference)
import functools

import jax
import jax.numpy as jnp
from jax import lax
from jax.experimental import pallas as pl
from jax.experimental.pallas import tpu as pltpu
from jax.experimental.pallas import tpu_sc as plsc

N_ATOM = 10000
N_BOND = 10000
K = 32
D = 128

NB = 80
NBLK = N_BOND // NB
NCH = NBLK * K

_info = plsc.get_sparse_core_info()
NC, NS = _info.num_cores, _info.num_subcores
NW = NC * NS
BPW = -(-NBLK // NW)
CH = 144
CL = 112
HEAVY_CORE = 1
NCHP = NS * (CH + CL) + CH - CL


@functools.partial(
    pl.kernel,
    mesh=plsc.VectorSubcoreMesh(core_axis_name="c", subcore_axis_name="s"),
    out_type=jax.ShapeDtypeStruct((N_BOND, (K + 1) * D), jnp.float32),
    scratch_types=[
        pltpu.VMEM((CH, NB), jnp.int32),
        pltpu.VMEM((8, NB, D), jnp.float32),
        pltpu.SemaphoreType.DMA,
        pltpu.SemaphoreType.DMA,
        pltpu.SemaphoreType.DMA,
        pltpu.SemaphoreType.DMA,
        pltpu.SemaphoreType.DMA,
        pltpu.SemaphoreType.DMA,
        pltpu.SemaphoreType.DMA,
        pltpu.SemaphoreType.DMA,
        pltpu.SemaphoreType.DMA,
        pltpu.SemaphoreType.DMA,
        pltpu.SemaphoreType.DMA,
        pltpu.SemaphoreType.DMA,
        pltpu.SemaphoreType.DMA,
        pltpu.SemaphoreType.DMA,
        pltpu.SemaphoreType.DMA,
        pltpu.SemaphoreType.DMA,
        pltpu.SemaphoreType.DMA,
    ],
)
def _concat_pool(atom_hbm, bond_hbm, idx_hbm, out_hbm, idx_v, rows_v,
                 g0, g1, g2, g3, g4, g5, g6, g7,
                 s0, s1, s2, s3, s4, s5, s6, s7, bsem):
    sid = lax.axis_index("s")
    cid = lax.axis_index("c")
    wid = sid * NC + cid
    heavy = cid == HEAVY_CORE
    ch0 = sid * (CH + CL) + jnp.where(heavy, 0, CH)
    cpw_w = jnp.where(heavy, CH, CL)
    blk0 = wid * BPW
    pltpu.sync_copy(idx_hbm.at[pl.ds(ch0, CH)], idx_v)

    def b_copy(i):
        blk = blk0 + i
        r0 = pl.multiple_of(blk * NB, NB)
        return pltpu.make_async_copy(
            bond_hbm.at[pl.ds(r0, NB)],
            out_hbm.at[pl.ds(r0, NB), pl.ds(0, D)], bsem)

    for i in range(BPW):
        @pl.when(blk0 + i < NBLK)
        def _():
            b_copy(i).start()

    gs = (g0, g1, g2, g3, g4, g5, g6, g7)
    ss = (s0, s1, s2, s3, s4, s5, s6, s7)

    def g_copy(j, b):
        return pltpu.make_async_copy(
            atom_hbm.at[idx_v.at[j]], rows_v.at[b], gs[b])

    def s_copy(j, b):
        c = ch0 + j
        blk = c // K
        t = c % K
        r0 = pl.multiple_of(blk * NB, NB)
        c0 = pl.multiple_of((t + 1) * D, D)
        return pltpu.make_async_copy(
            rows_v.at[b], out_hbm.at[pl.ds(r0, NB), pl.ds(c0, D)], ss[b])

    def live(j):
        return (j >= 0) & (j < cpw_w) & (ch0 + j < NCH)

    def start(j, b):
        @pl.when(live(j))
        def _():
            g_copy(j, b).start()

    def finish(j, b):
        @pl.when(live(j))
        def _():
            g_copy(j, b).wait()
            s_copy(j, b).start()

    def drain(j, b):
        @pl.when(live(j))
        def _():
            s_copy(j, b).wait()

    for b in range(8):
        start(b, b)

    def body(q, carry):
        j = 8 * q
        for b in range(8):
            finish(j + b, b)
        for b in range(8):
            drain(j + b, b)
            start(j + b + 8, b)
        return carry

    lax.fori_loop(0, CH // 8, body, 0)

    for i in range(BPW):
        @pl.when(blk0 + i < NBLK)
        def _():
            b_copy(i).wait()


def kernel(atom_ft, bond_ft, edge_index):
    src = edge_index[0]
    srcp = jnp.pad(src.reshape(N_BOND, K), ((0, NCHP // K * NB - N_BOND), (0, 0)))
    idx = srcp.reshape(NCHP // K, NB, K).transpose(0, 2, 1).reshape(NCHP, NB)
    out = _concat_pool(atom_ft, bond_ft, idx)
    return (atom_ft, out)


# speedup vs baseline: 1.0027x; 1.0027x over previous
import functools

import jax
import jax.numpy as jnp
from jax import lax
from jax.experimental import pallas as pl
from jax.experimental.pallas import tpu as pltpu
from jax.experimental.pallas import tpu_sc as plsc

N_ATOM = 10000
N_BOND = 10000
K = 32
D = 128

NB = 80
NBLK = N_BOND // NB
NCH = NBLK * K

_info = plsc.get_sparse_core_info()
NC, NS = _info.num_cores, _info.num_subcores
NW = NC * NS
BPW = -(-NBLK // NW)
CPW = BPW * K


@functools.partial(
    pl.kernel,
    mesh=plsc.VectorSubcoreMesh(core_axis_name="c", subcore_axis_name="s"),
    out_type=jax.ShapeDtypeStruct((N_BOND, (K + 1) * D), jnp.float32),
    scratch_types=[
        pltpu.VMEM((CPW, NB), jnp.int32),
        pltpu.VMEM((8, NB, D), jnp.float32),
        pltpu.SemaphoreType.DMA,
        pltpu.SemaphoreType.DMA,
        pltpu.SemaphoreType.DMA,
        pltpu.SemaphoreType.DMA,
        pltpu.SemaphoreType.DMA,
        pltpu.SemaphoreType.DMA,
        pltpu.SemaphoreType.DMA,
        pltpu.SemaphoreType.DMA,
        pltpu.SemaphoreType.DMA,
        pltpu.SemaphoreType.DMA,
        pltpu.SemaphoreType.DMA,
        pltpu.SemaphoreType.DMA,
        pltpu.SemaphoreType.DMA,
        pltpu.SemaphoreType.DMA,
        pltpu.SemaphoreType.DMA,
        pltpu.SemaphoreType.DMA,
        pltpu.SemaphoreType.DMA,
    ],
)
def _concat_pool(atom_hbm, bond_hbm, idx_hbm, out_hbm, idx_v, rows_v,
                 g0, g1, g2, g3, g4, g5, g6, g7,
                 s0, s1, s2, s3, s4, s5, s6, s7, bsem):
    wid = lax.axis_index("s") * NC + lax.axis_index("c")
    ch0 = wid * CPW
    blk0 = wid * BPW
    pltpu.sync_copy(idx_hbm.at[pl.ds(ch0, 32)], idx_v.at[pl.ds(0, 32)])

    @pl.when(ch0 + CPW <= NCH)
    def _():
        pltpu.sync_copy(idx_hbm.at[pl.ds(ch0 + 32, CPW - 32)],
                        idx_v.at[pl.ds(32, CPW - 32)])

    def b_copy(i):
        blk = blk0 + i
        r0 = pl.multiple_of(blk * NB, NB)
        return pltpu.make_async_copy(
            bond_hbm.at[pl.ds(r0, NB)],
            out_hbm.at[pl.ds(r0, NB), pl.ds(0, D)], bsem)

    for i in range(BPW):
        @pl.when(blk0 + i < NBLK)
        def _():
            b_copy(i).start()

    gs = (g0, g1, g2, g3, g4, g5, g6, g7)
    ss = (s0, s1, s2, s3, s4, s5, s6, s7)

    def g_copy(j, b):
        return pltpu.make_async_copy(
            atom_hbm.at[idx_v.at[j]], rows_v.at[b], gs[b])

    def s_copy(j, b):
        c = ch0 + j
        blk = c // K
        t = c % K
        r0 = pl.multiple_of(blk * NB, NB)
        c0 = pl.multiple_of((t + 1) * D, D)
        return pltpu.make_async_copy(
            rows_v.at[b], out_hbm.at[pl.ds(r0, NB), pl.ds(c0, D)], ss[b])

    def live(j):
        return (j >= 0) & (j < CPW) & (ch0 + j < NCH)

    def start(j, b):
        @pl.when(live(j))
        def _():
            g_copy(j, b).start()

    def finish(j, b):
        @pl.when(live(j))
        def _():
            g_copy(j, b).wait()
            s_copy(j, b).start()

    def drain(j, b):
        @pl.when(live(j))
        def _():
            s_copy(j, b).wait()

    for b in range(8):
        start(b, b)

    def body(q, carry):
        j = 8 * q
        for b in range(8):
            finish(j + b, b)
        for b in range(8):
            drain(j + b, b)
            start(j + b + 8, b)
        return carry

    lax.fori_loop(0, CPW // 8, body, 0)

    for i in range(BPW):
        @pl.when(blk0 + i < NBLK)
        def _():
            b_copy(i).wait()


def kernel(atom_ft, bond_ft, edge_index):
    src = edge_index[0]
    idx = src.reshape(NBLK, NB, K).transpose(0, 2, 1).reshape(NCH, NB)
    out = _concat_pool(atom_ft, bond_ft, idx)
    return (atom_ft, out)
